# Initial kernel scaffold; baseline (speedup 1.0000x reference)
import jax
import jax.numpy as jnp
from jax.experimental import pallas as pl


def kernel(x, params, edge_index, head):
    raise NotImplementedError("write your pallas kernel here")


# speedup vs baseline: 15.7137x; 15.7137x over previous
import functools

import jax
import jax.numpy as jnp
from jax import lax
from jax.experimental import pallas as pl
from jax.experimental.pallas import tpu as pltpu
from jax.experimental.pallas import tpu_sc as plsc

N = 10000
E = 320000
NFEAT = 128
NHID = 128
NHEADS = 3
D = 144
NPAD = 10240
NC = 2
NS = 16
EPW = E // (NC * NS)
CH = 80
NCHUNK = EPW // CH
RPT = NPAD // NS
ZR = 16


def _leaky(x):
    return jnp.where(x >= 0, x, 0.2 * x)


def _s1_body(xpad, src, dst, zin, out, gidx, sidx, rows, acc, sem):
    cid = lax.axis_index("c")
    sid = lax.axis_index("s")
    wid = cid * NS + sid

    @pl.loop(0, RPT // ZR)
    def _zero(z):
        pltpu.sync_copy(zin, acc.at[pl.ds(sid * RPT + z * ZR, ZR)])

    plsc.subcore_barrier()

    def chunk(k):
        eb = wid * EPW + k * CH
        pltpu.sync_copy(src.at[pl.ds(eb, CH)], gidx)
        pltpu.sync_copy(dst.at[pl.ds(eb, CH)], sidx)
        pltpu.async_copy(xpad.at[gidx], rows, sem).wait()
        pltpu.sync_copy(rows, acc.at[sidx], add=True)

    pl.loop(0, NCHUNK)(chunk)
    plsc.subcore_barrier()
    pltpu.sync_copy(acc.at[pl.ds(sid * RPT, RPT)],
                    out.at[cid, pl.ds(sid * RPT, RPT)])


@functools.partial(
    pl.kernel,
    out_type=jax.ShapeDtypeStruct((NC, NPAD, D), jnp.float32),
    mesh=plsc.VectorSubcoreMesh(core_axis_name="c", subcore_axis_name="s"),
    compiler_params=pltpu.CompilerParams(use_tc_tiling_on_sc=False, needs_layout_passes=False),
    scratch_types=[
        pltpu.VMEM((CH,), jnp.int32),
        pltpu.VMEM((CH,), jnp.int32),
        pltpu.VMEM((CH, D), jnp.float32),
        pltpu.VMEM_SHARED((NPAD, D), jnp.float32),
        pltpu.SemaphoreType.DMA,
    ],
)
def _s1(xpad, src, dst, zin, out, gidx, sidx, rows, acc, sem):
    _s1_body(xpad, src, dst, zin, out, gidx, sidx, rows, acc, sem)


def _s3_body(hx, asrc, adst, cv, srcoff, src, dst, zin, out,
             gidx, goff, sidx, ev, rows, av, dv, cvv, acc, sem):
    cid = lax.axis_index("c")
    sid = lax.axis_index("s")
    wid = cid * NS + sid

    pltpu.sync_copy(cv, cvv)

    @pl.loop(0, RPT // ZR)
    def _zero0(z):
        pltpu.sync_copy(zin, acc.at[pl.ds(sid * RPT + z * ZR, ZR)])

    plsc.subcore_barrier()

    zof = (cid - cid) + (sid - sid)
    zv16 = jnp.full((16,), zof, jnp.int32)

    for h in range(NHEADS):
        pltpu.sync_copy(asrc.at[h], av)
        pltpu.sync_copy(adst.at[h], dv)
        cvv16 = cvv[...]
        hmask = lax.iota(jnp.int32, 16) == h
        csp = jnp.sum(jnp.where(hmask, cvv16, 0.0), axis=0)

        @pl.loop(0, RPT // ZR)
        def _zero(z):
            pltpu.sync_copy(zin, acc.at[pl.ds(sid * RPT + z * ZR, ZR)])

        plsc.subcore_barrier()

        def chunk(k):
            eb = wid * EPW + k * CH
            pltpu.sync_copy(srcoff.at[h, pl.ds(eb, CH)], goff)
            pltpu.sync_copy(src.at[pl.ds(eb, CH)], gidx)
            pltpu.sync_copy(dst.at[pl.ds(eb, CH)], sidx)
            pltpu.sync_copy(hx.at[goff], rows)
            lane16 = lax.iota(jnp.int32, 16)
            for j in range(CH // 16):
                s16 = gidx[pl.ds(16 * j, 16)]
                d16 = sidx[pl.ds(16 * j, 16)]
                logit = (plsc.load_gather(av, [s16])
                         + plsc.load_gather(dv, [d16]))
                logit = jnp.where(logit >= 0, logit, 0.2 * logit)
                e16 = jnp.exp(logit - csp)
                for r2 in range(16):
                    es = jnp.sum(jnp.where(lane16 == r2, e16, 0.0), axis=0)
                    r = 16 * j + r2
                    for q in range(D // 16):
                        rows[r, pl.ds(16 * q, 16)] = (
                            rows[r, pl.ds(16 * q, 16)] * es)
            pltpu.sync_copy(rows, acc.at[sidx], add=True)

        pl.loop(0, NCHUNK)(chunk)
        plsc.subcore_barrier()
        pltpu.sync_copy(acc.at[pl.ds(sid * RPT, RPT)],
                        out.at[h * NC + cid, pl.ds(sid * RPT, RPT)])
        plsc.subcore_barrier()


@functools.partial(
    pl.kernel,
    out_type=jax.ShapeDtypeStruct((NHEADS * NC, NPAD, D), jnp.float32),
    mesh=plsc.VectorSubcoreMesh(core_axis_name="c", subcore_axis_name="s"),
    compiler_params=pltpu.CompilerParams(use_tc_tiling_on_sc=False, needs_layout_passes=False),
    scratch_types=[
        pltpu.VMEM((CH,), jnp.int32),
        pltpu.VMEM((CH,), jnp.int32),
        pltpu.VMEM((CH,), jnp.int32),
        pltpu.VMEM((CH,), jnp.float32),
        pltpu.VMEM((CH, D), jnp.float32),
        pltpu.VMEM((N,), jnp.float32),
        pltpu.VMEM((N,), jnp.float32),
        pltpu.VMEM((16,), jnp.float32),
        pltpu.VMEM_SHARED((NPAD, D), jnp.float32),
        pltpu.SemaphoreType.DMA,
    ],
)
def _s3(hx, asrc, adst, cv, srcoff, src, dst, zin, out,
        gidx, goff, sidx, ev, rows, av, dv, cvv, acc, sem):
    _s3_body(hx, asrc, adst, cv, srcoff, src, dst, zin, out,
             gidx, goff, sidx, ev, rows, av, dv, cvv, acc, sem)


def _t1_kernel(x_ref, p_ref, g1, g2, b1, b2, r_ref, w_ref, as_ref, ad_ref,
               head_ref, out_ref, hx_ref, asrc_ref, adst_ref, cv_ref, max_ref):
    nb = pl.program_id(0)
    nblk = pl.num_programs(0)
    bn = x_ref.shape[0]
    x = x_ref[...]
    psum = p_ref[0] + p_ref[1]
    deg = jnp.maximum(psum[:, NFEAT:NFEAT + 1], 1.0)
    neighbor = psum[:, :NFEAT] / deg
    gamma = _leaky(jnp.dot(x, g1[...].T, preferred_element_type=jnp.float32)
                   + jnp.dot(neighbor, g2[...].T,
                             preferred_element_type=jnp.float32)) + 1.0
    beta = _leaky(jnp.dot(x, b1[...].T, preferred_element_type=jnp.float32)
                  + jnp.dot(neighbor, b2[...].T,
                            preferred_element_type=jnp.float32))
    output = x + gamma * r_ref[...] + beta - neighbor
    out_ref[...] = output
    xin = jnp.where(head_ref[0, 0] != 0, x, x + output)

    @pl.when(nb == 0)
    def _():
        max_ref[...] = jnp.full((8, 128), -jnp.inf, jnp.float32)

    pad_col = lax.broadcasted_iota(jnp.int32, (bn, D - NFEAT), 1)
    rows8 = lax.broadcasted_iota(jnp.int32, (8, 128), 0)
    lanes8 = lax.broadcasted_iota(jnp.int32, (8, 128), 1)
    for i in range(NHEADS):
        h = jnp.dot(xin, w_ref[i], preferred_element_type=jnp.float32)
        hx_ref[i, :, :NFEAT] = h
        hx_ref[i, :, NFEAT:] = jnp.where(pad_col == 0, 1.0, 0.0)
        asrc = jnp.dot(h, as_ref[i][:, None],
                       preferred_element_type=jnp.float32)
        adst = jnp.dot(h, ad_ref[i][:, None],
                       preferred_element_type=jnp.float32)
        asrc_ref[:, i:i + 1] = asrc
        adst_ref[:, i:i + 1] = adst
        ms = jnp.max(asrc)
        md = jnp.max(adst)
        upd = jnp.where((rows8 == 0) & (lanes8 == i), ms,
                        jnp.where((rows8 == 1) & (lanes8 == i), md, -jnp.inf))
        max_ref[...] = jnp.maximum(max_ref[...], upd)

    @pl.when(nb == nblk - 1)
    def _():
        m = max_ref[...]
        c = _leaky(m[0:1, :] + m[1:2, :])
        lane = lax.broadcasted_iota(jnp.int32, (1, 128), 1)
        cv_ref[...] = jnp.where(lane < NHEADS, c, 0.0)


def _t3_kernel(p_ref, hx_ref, asrc_ref, adst_ref, cv_ref, bias_ref, out_ref):
    lane = lax.broadcasted_iota(jnp.int32, (1, 128), 1)
    cv = cv_ref[...]
    for i in range(NHEADS):
        acc = p_ref[i, 0] + p_ref[i, 1]
        num = acc[:, :NFEAT]
        den = acc[:, NFEAT:NFEAT + 1]
        h = hx_ref[i, :, :NFEAT]
        c_i = jnp.sum(jnp.where(lane == i, cv, 0.0))
        a_blk = asrc_ref[:, i:i + 1] + adst_ref[:, i:i + 1]
        selfe = jnp.exp(_leaky(a_blk) - c_i)
        num = num + selfe * h
        den = den + selfe
        out_ref[:, i * NHID:(i + 1) * NHID] = (
            num / (den + 1e-16) + bias_ref[i][None, :])


def kernel(x, params, edge_index, head):
    src = edge_index[0]
    dst = edge_index[1]
    rel = params['rel']
    gats = params['gat']

    xpad = jnp.concatenate(
        [x, jnp.ones((N, 1), x.dtype), jnp.zeros((N, D - NFEAT - 1), x.dtype)],
        axis=1)

    zin = jnp.zeros((ZR, D), jnp.float32)

    nb_part = _s1(xpad, dst, src, zin)

    w = jnp.stack([g['W'] for g in gats])
    a_src = jnp.stack([g['a_src'] for g in gats])
    a_dst = jnp.stack([g['a_dst'] for g in gats])
    bias = jnp.stack([g['bias'] for g in gats])
    head_arr = jnp.asarray(head, jnp.int32).reshape(1, 1)

    BT = 2000
    output, hx, asrc, adst, cv = pl.pallas_call(
        _t1_kernel,
        grid=(N // BT,),
        out_shape=(
            jax.ShapeDtypeStruct((N, NFEAT), jnp.float32),
            jax.ShapeDtypeStruct((NHEADS, N, D), jnp.float32),
            jax.ShapeDtypeStruct((N, NHEADS), jnp.float32),
            jax.ShapeDtypeStruct((N, NHEADS), jnp.float32),
            jax.ShapeDtypeStruct((1, 128), jnp.float32),
        ),
        in_specs=[
            pl.BlockSpec((BT, NFEAT), lambda n: (n, 0)),
            pl.BlockSpec((NC, BT, D), lambda n: (0, n, 0)),
            pl.BlockSpec((NFEAT, NFEAT), lambda n: (0, 0)),
            pl.BlockSpec((NFEAT, NFEAT), lambda n: (0, 0)),
            pl.BlockSpec((NFEAT, NFEAT), lambda n: (0, 0)),
            pl.BlockSpec((NFEAT, NFEAT), lambda n: (0, 0)),
            pl.BlockSpec((1, NFEAT), lambda n: (0, 0)),
            pl.BlockSpec((NHEADS, NFEAT, NHID), lambda n: (0, 0, 0)),
            pl.BlockSpec((NHEADS, NHID), lambda n: (0, 0)),
            pl.BlockSpec((NHEADS, NHID), lambda n: (0, 0)),
            pl.BlockSpec((1, 1), lambda n: (0, 0), memory_space=pltpu.SMEM),
        ],
        out_specs=[
            pl.BlockSpec((BT, NFEAT), lambda n: (n, 0)),
            pl.BlockSpec((NHEADS, BT, D), lambda n: (0, n, 0)),
            pl.BlockSpec((BT, NHEADS), lambda n: (n, 0)),
            pl.BlockSpec((BT, NHEADS), lambda n: (n, 0)),
            pl.BlockSpec((1, 128), lambda n: (0, 0)),
        ],
        scratch_shapes=[pltpu.VMEM((8, 128), jnp.float32)],
    )(x, nb_part[:, :N], rel['g1'], rel['g2'], rel['b1'], rel['b2'], rel['r'],
      w, a_src, a_dst, head_arr)

    asrc_cols, adst_cols = asrc, adst
    asrc = asrc_cols.T
    adst = adst_cols.T
    hx_flat = hx.reshape(NHEADS * N, D)
    srcoff = src[None, :] + (jnp.arange(NHEADS, dtype=jnp.int32) * N)[:, None]

    gat_part = _s3(hx_flat, asrc, adst, cv[0, :16], srcoff, src, dst, zin)
    gat_part = gat_part.reshape(NHEADS, NC, NPAD, D)

    BN = 2000
    h_k = pl.pallas_call(
        _t3_kernel,
        grid=(N // BN,),
        out_shape=jax.ShapeDtypeStruct((N, NHEADS * NHID), jnp.float32),
        in_specs=[
            pl.BlockSpec((NHEADS, NC, BN, D), lambda n: (0, 0, n, 0)),
            pl.BlockSpec((NHEADS, BN, D), lambda n: (0, n, 0)),
            pl.BlockSpec((BN, NHEADS), lambda n: (n, 0)),
            pl.BlockSpec((BN, NHEADS), lambda n: (n, 0)),
            pl.BlockSpec((1, 128), lambda n: (0, 0)),
            pl.BlockSpec((NHEADS, 128), lambda n: (0, 0)),
        ],
        out_specs=pl.BlockSpec((BN, NHEADS * NHID), lambda n: (n, 0)),
    )(gat_part, hx, asrc_cols, adst_cols, cv, bias)

    return (h_k, output)
